# Initial kernel scaffold; baseline (speedup 1.0000x reference)
import jax
import jax.numpy as jnp
from jax.experimental import pallas as pl


def kernel(token_ids, weight):
    raise NotImplementedError("write your pallas kernel here")


# speedup vs baseline: 3.3415x; 3.3415x over previous
import functools

import jax
import jax.numpy as jnp
from jax import lax
from jax.experimental import pallas as pl
from jax.experimental.pallas import tpu as pltpu
from jax.experimental.pallas import tpu_sc as plsc

NUM_SUBCORES = 16
NUM_CORES = 2
NW = NUM_CORES * NUM_SUBCORES

CHUNK = 128
NBUF = 5


@functools.cache
def _build(n_rows, vocab, d):
    n_chunks = n_rows // NW
    groups = n_chunks // NBUF
    mesh = plsc.VectorSubcoreMesh(core_axis_name="c", subcore_axis_name="s")

    def body(idx_hbm, table_hbm, out_hbm, idx_v, *rest):
        bufs = rest[:NBUF]
        gsems = rest[NBUF:2 * NBUF]
        ssems = rest[2 * NBUF:]

        wid = lax.axis_index("c") * NUM_SUBCORES + lax.axis_index("s")
        id0 = wid * n_chunks * CHUNK

        pltpu.sync_copy(idx_hbm.at[pl.ds(id0, n_chunks * CHUNK)], idx_v)

        def start_gather(j, b):
            pltpu.async_copy(
                table_hbm.at[idx_v.at[pl.ds(j * CHUNK, CHUNK)]],
                bufs[b], gsems[b])

        def wait_gather(j, b):
            pltpu.make_async_copy(
                table_hbm.at[idx_v.at[pl.ds(j * CHUNK, CHUNK)]],
                bufs[b], gsems[b]).wait()

        def start_scatter(j, b):
            pltpu.async_copy(
                bufs[b], out_hbm.at[pl.ds(id0 + j * CHUNK, CHUNK)], ssems[b])

        def wait_scatter(j, b):
            pltpu.make_async_copy(
                bufs[b], out_hbm.at[pl.ds(id0 + j * CHUNK, CHUNK)],
                ssems[b]).wait()

        for b in range(NBUF):
            start_gather(b, b)

        @pl.loop(0, groups - 1)
        def _(g):
            for b in range(NBUF):
                j = g * NBUF + b
                wait_gather(j, b)
                start_scatter(j, b)
                wait_scatter(j, b)
                start_gather(j + NBUF, b)

        for b in range(NBUF):
            j = (groups - 1) * NBUF + b
            wait_gather(j, b)
            start_scatter(j, b)
        for b in range(NBUF):
            j = (groups - 1) * NBUF + b
            wait_scatter(j, b)

    run = pl.kernel(
        body,
        out_type=jax.ShapeDtypeStruct((n_rows * CHUNK, d), jnp.float32),
        mesh=mesh,
        scratch_types=(
            [pltpu.VMEM((n_chunks * CHUNK,), jnp.int32)]
            + [pltpu.VMEM((CHUNK, d), jnp.float32) for _ in range(NBUF)]
            + [pltpu.SemaphoreType.DMA for _ in range(2 * NBUF)]
        ),
    )
    return run


def kernel(token_ids, weight):
    bsz, seq = token_ids.shape
    vocab, d = weight.shape
    total = bsz * seq
    idx1d = token_ids.astype(jnp.int32).reshape(total)
    out = _build(total // CHUNK, vocab, d)(idx1d, weight)
    return out.reshape(bsz, seq, d)
